# baseline (speedup 1.0000x reference)
import functools

import jax
import jax.numpy as jnp
from jax import lax
from jax.experimental import pallas as pl
from jax.experimental.pallas import tpu as pltpu
from jax.experimental.pallas import tpu_sc as plsc

F32 = jnp.float32
_NT = 16
_SUP = 1024
_IDXG = 128


def _ln_ref(x, g, b):
    mean = jnp.mean(x, axis=-1, keepdims=True)
    var = jnp.var(x, axis=-1, keepdims=True)
    return (x - mean) / jnp.sqrt(var + 1e-5) * g + b


def _relu(x):
    return jnp.maximum(x, 0.0)


def _dot(x, w):
    return jnp.dot(x.astype(jnp.bfloat16), w.astype(jnp.bfloat16),
                   preferred_element_type=F32)


def _enc_body(x_ref, w0, b0, w1, b1, w2, b2, o_ref):
    h = _relu(_dot(x_ref[...], w0[...]) + b0[...])
    h = _relu(_dot(h, w1[...]) + b1[...])
    o_ref[...] = _dot(h, w2[...]) + b2[...]


def _full(shape):
    return pl.BlockSpec(shape, lambda i: tuple(0 for _ in shape))


def _encoder(x, p, rb):
    n, din = x.shape
    w0, w1, w2 = p["W"]
    b0, b1, b2 = (v.reshape(1, -1) for v in p["b"])
    dout = w2.shape[1]
    h = pl.pallas_call(
        _enc_body,
        grid=(n // rb,),
        in_specs=[
            pl.BlockSpec((rb, din), lambda i: (i, 0)),
            _full(w0.shape), _full(b0.shape), _full(w1.shape), _full(b1.shape),
            _full(w2.shape), _full(b2.shape),
        ],
        out_specs=pl.BlockSpec((rb, dout), lambda i: (i, 0)),
        out_shape=jax.ShapeDtypeStruct((n, dout), F32),
    )(x, w0, b0, w1, b1, w2, b2)
    return _ln_ref(h, p["ln_g"], p["ln_b"])


def _edge_body(e_ref, r_ref, s_ref, w0, b0, w1, b1, w2, b2, h_ref):
    x = jnp.concatenate(
        [e_ref[...].astype(jnp.bfloat16), r_ref[...], s_ref[...]], axis=1)
    h = _relu(jnp.dot(x, w0[...].astype(jnp.bfloat16),
                      preferred_element_type=F32) + b0[...])
    h = _relu(_dot(h, w1[...]) + b1[...])
    h_ref[...] = _dot(h, w2[...]) + b2[...]


def _edge_stage(edge, r_rows, s_rows, p, rb):
    n, d = edge.shape
    w0, w1, w2 = p["W"]
    b0, b1, b2 = (v.reshape(1, -1) for v in p["b"])
    h = pl.pallas_call(
        _edge_body,
        grid=(n // rb,),
        in_specs=[
            pl.BlockSpec((rb, d), lambda i: (i, 0)),
            pl.BlockSpec((rb, d), lambda i: (i, 0)),
            pl.BlockSpec((rb, d), lambda i: (i, 0)),
            _full(w0.shape), _full(b0.shape), _full(w1.shape), _full(b1.shape),
            _full(w2.shape), _full(b2.shape),
        ],
        out_specs=pl.BlockSpec((rb, d), lambda i: (i, 0)),
        out_shape=jax.ShapeDtypeStruct((n, d), F32),
    )(edge, r_rows, s_rows, w0, b0, w1, b1, w2, b2)
    proc = _ln_ref(h, p["ln_g"], p["ln_b"])
    return proc, edge + proc


def _node_body(n_ref, a_ref, gl_ref, w0, b0, w1, b1, w2, b2, h_ref):
    x = jnp.concatenate(
        [n_ref[...].astype(jnp.bfloat16), a_ref[...].astype(jnp.bfloat16),
         gl_ref[...].astype(jnp.bfloat16)], axis=1)
    h = _relu(jnp.dot(x, w0[...].astype(jnp.bfloat16),
                      preferred_element_type=F32) + b0[...])
    h = _relu(_dot(h, w1[...]) + b1[...])
    h_ref[...] = _dot(h, w2[...]) + b2[...]


def _node_stage(node, agg, glob, p, rb):
    n, d = node.shape
    w0, w1, w2 = p["W"]
    b0, b1, b2 = (v.reshape(1, -1) for v in p["b"])
    h = pl.pallas_call(
        _node_body,
        grid=(n // rb,),
        in_specs=[
            pl.BlockSpec((rb, d), lambda i: (i, 0)),
            pl.BlockSpec((rb, d), lambda i: (i, 0)),
            pl.BlockSpec((rb, glob.shape[1]), lambda i: (i, 0)),
            _full(w0.shape), _full(b0.shape), _full(w1.shape), _full(b1.shape),
            _full(w2.shape), _full(b2.shape),
        ],
        out_specs=pl.BlockSpec((rb, d), lambda i: (i, 0)),
        out_shape=jax.ShapeDtypeStruct((n, d), F32),
    )(node, agg, glob, w0, b0, w1, b1, w2, b2)
    return node + _ln_ref(h, p["ln_g"], p["ln_b"])


def _dec_body(x_ref, w0, b0, w1, b1, w2, b2, o_ref):
    h = _relu(_dot(x_ref[...], w0[...]) + b0[...])
    h = _relu(_dot(h, w1[...]) + b1[...])
    o_ref[...] = _dot(h, w2[...]) + b2[...]


def _decoder(x, p, rb):
    n, _ = x.shape
    w0, w1, w2 = p["W"]
    b0, b1, b2 = (v.reshape(1, -1) for v in p["b"])
    dout = w2.shape[1]
    return pl.pallas_call(
        _dec_body,
        grid=(n // rb,),
        in_specs=[
            pl.BlockSpec((rb, x.shape[1]), lambda i: (i, 0)),
            _full(w0.shape), _full(b0.shape), _full(w1.shape), _full(b1.shape),
            _full(w2.shape), _full(b2.shape),
        ],
        out_specs=pl.BlockSpec((rb, dout), lambda i: (i, 0)),
        out_shape=jax.ShapeDtypeStruct((n, dout), F32),
    )(x, w0, b0, w1, b1, w2, b2)


@functools.lru_cache(maxsize=None)
def _gather_kernel(n_nodes, e, d):
    per = e // _NT
    nf, tail = divmod(per, _SUP)
    ntg, tg_rem = divmod(tail, _IDXG)
    assert nf >= 4 and nf % 2 == 0 and tail % 16 == 0
    mesh = plsc.VectorSubcoreMesh(core_axis_name="c", subcore_axis_name="s")
    out = jax.ShapeDtypeStruct((e, d), jnp.bfloat16)

    @functools.partial(
        pl.kernel,
        out_type=(out, out),
        mesh=mesh,
        compiler_params=pltpu.CompilerParams(use_tc_tiling_on_sc=False),
        scratch_types=[
            pltpu.VMEM((2, _SUP), jnp.int32),
            pltpu.VMEM((2, _SUP, d), jnp.bfloat16),
            pltpu.SemaphoreType.DMA,
            pltpu.SemaphoreType.DMA,
            pltpu.SemaphoreType.DMA,
        ],
    )
    def k(tab, ridx, sidx, rout, sout, idx_v, rows_v, gsem, wsem, isem):
        c = lax.axis_index("c")
        s = lax.axis_index("s")
        base = s * per

        def run(ih, oh):
            def istart(i, b, n):
                pltpu.async_copy(ih.at[pl.ds(base + i * _SUP, n)],
                                 idx_v.at[b, pl.ds(0, n)], isem)

            def iwait(b, n):
                pltpu.make_async_copy(ih.at[pl.ds(base, n)],
                                      idx_v.at[b, pl.ds(0, n)], isem).wait()

            def wwait(b, n):
                pltpu.make_async_copy(rows_v.at[b, pl.ds(0, n)],
                                      oh.at[pl.ds(base, n)], wsem).wait()

            def fire(i, b, ngroups, rem):
                cps = [
                    pltpu.async_copy(
                        tab.at[idx_v.at[b, pl.ds(gi * _IDXG, _IDXG)]],
                        rows_v.at[b, pl.ds(gi * _IDXG, _IDXG)], gsem)
                    for gi in range(ngroups)
                ]
                if rem:
                    cps.append(pltpu.async_copy(
                        tab.at[idx_v.at[b, pl.ds(ngroups * _IDXG, rem)]],
                        rows_v.at[b, pl.ds(ngroups * _IDXG, rem)], gsem))
                for cp in cps:
                    cp.wait()
                n = ngroups * _IDXG + rem
                pltpu.async_copy(rows_v.at[b, pl.ds(0, n)],
                                 oh.at[pl.ds(base + i * _SUP, n)], wsem)

            ng = _SUP // _IDXG
            istart(0, 0, _SUP)
            iwait(0, _SUP)
            istart(1, 1, _SUP)
            fire(0, 0, ng, 0)
            iwait(1, _SUP)
            istart(2, 0, _SUP)
            fire(1, 1, ng, 0)

            def pair(t, _):
                for b in (0, 1):
                    i = 2 + 2 * t + b
                    iwait(b, _SUP)
                    istart(i + 1, 1 - b, _SUP)
                    wwait(b, _SUP)
                    fire(i, b, ng, 0)
                return 0

            lax.fori_loop(0, (nf - 4) // 2, pair, 0)

            iwait(0, _SUP)
            istart(nf - 1, 1, _SUP)
            wwait(0, _SUP)
            fire(nf - 2, 0, ng, 0)
            iwait(1, _SUP)
            if tail:
                istart(nf, 0, tail)
            wwait(1, _SUP)
            fire(nf - 1, 1, ng, 0)
            if tail:
                iwait(0, tail)
                wwait(0, _SUP)
                fire(nf, 0, ntg, tg_rem)
            wwait(1, _SUP)
            if tail:
                wwait(0, tail)

        pl.when(c == 0)(lambda: run(ridx, rout))
        pl.when(c == 1)(lambda: run(sidx, sout))

    return k


def _tile_node_starts(n_nodes):
    half = n_nodes // 2
    q, r = divmod(half, _NT)
    starts = []
    for c in (0, 1):
        for s in range(_NT):
            off = s * (q + 1) if s < r else r * (q + 1) + (s - r) * q
            starts.append(c * half + off)
    return starts


@functools.lru_cache(maxsize=None)
def _scatter_kernel(n_nodes, e, d):
    chunk = 368
    ext = 384
    half = n_nodes // 2
    pad = ((half + 8 + 15) // 16) * 16
    zrows = pad // _NT
    q, r = divmod(half, _NT)
    mesh = plsc.VectorSubcoreMesh(core_axis_name="c", subcore_axis_name="s")

    @functools.partial(
        pl.kernel,
        out_type=jax.ShapeDtypeStruct((n_nodes, d), F32),
        mesh=mesh,
        compiler_params=pltpu.CompilerParams(use_tc_tiling_on_sc=False),
        scratch_types=[
            pltpu.VMEM((ext,), jnp.int32),
            pltpu.VMEM((ext // _IDXG, _IDXG), jnp.int32),
            pltpu.VMEM((ext, d), F32),
            pltpu.VMEM_SHARED((pad, d), F32),
            pltpu.VMEM((16,), jnp.int32),
            pltpu.SemaphoreType.DMA,
        ],
    )
    def k(idx_hbm, rows_hbm, zero_hbm, bounds_hbm, out_hbm,
          raw_v, idx2_v, rows_v, acc, bvm, sem):
        c = lax.axis_index("c")
        s = lax.axis_index("s")
        nbase = c * half
        t = c * _NT + s
        pltpu.sync_copy(bounds_hbm.at[t], bvm)
        pltpu.sync_copy(zero_hbm.at[pl.ds(s * zrows, zrows)],
                        acc.at[pl.ds(s * zrows, zrows)])
        plsc.subcore_barrier()

        lanes = lax.iota(jnp.int32, 16)
        bv = bvm[...]
        span_s = bv[0]
        span_e = bv[1]
        nch = lax.div(span_e - span_s + (chunk - 1), chunk)

        def body(kk, _):
            logical = span_s + kk * chunk
            lim = jnp.minimum(logical + chunk, span_e)
            off8 = lax.div(jnp.minimum(logical, e - ext), 8) * 8
            pltpu.sync_copy(idx_hbm.at[pl.ds(off8, ext)], raw_v)
            pltpu.sync_copy(rows_hbm.at[pl.ds(off8, ext)], rows_v)

            def a_body(j, _):
                v = raw_v[pl.ds(j * 16, 16)]
                gpos = off8 + j * 16 + lanes
                ok = (gpos >= logical) & (gpos < lim)
                lv = jnp.where(ok, v - nbase, half)
                idx2_v[j // 8, pl.ds((j % 8) * 16, 16)] = lv
                return 0

            lax.fori_loop(0, ext // 16, a_body, 0)
            for gi in range(ext // _IDXG):
                pltpu.sync_copy(rows_v.at[pl.ds(gi * _IDXG, _IDXG)],
                                acc.at[idx2_v.at[gi]], add=True)
            return 0

        lax.fori_loop(0, nch, body, 0)
        plsc.subcore_barrier()

        @pl.when(s < r)
        def _():
            a_off = s * (q + 1)
            pltpu.sync_copy(acc.at[pl.ds(a_off, q + 1)],
                            out_hbm.at[pl.ds(nbase + a_off, q + 1)])

        @pl.when(s >= r)
        def _():
            a_off = r * (q + 1) + (s - r) * q
            pltpu.sync_copy(acc.at[pl.ds(a_off, q)],
                            out_hbm.at[pl.ds(nbase + a_off, q)])

    return k


def kernel(node_feat, edge_feat, global_feat, params, edge_idx, node_size):
    n_nodes, _ = node_feat.shape
    n_edges = edge_idx.shape[0]

    perm = jnp.argsort(jnp.asarray(edge_idx[:, 0], jnp.int32), stable=True)
    ridx = jnp.take(jnp.asarray(edge_idx[:, 0], jnp.int32), perm)
    sidx = jnp.take(jnp.asarray(edge_idx[:, 1], jnp.int32), perm)
    edge_feat = jnp.take(edge_feat, perm, axis=0)

    starts = _tile_node_starts(n_nodes)
    bounds = jnp.searchsorted(
        ridx, jnp.asarray(starts + [n_nodes], jnp.int32)).astype(jnp.int32)
    bounds2 = jnp.zeros((2 * _NT, 16), jnp.int32)
    bounds2 = bounds2.at[:, 0].set(bounds[:-1]).at[:, 1].set(bounds[1:])

    node = _encoder(node_feat, params["node_enc"], 2000)
    edge = _encoder(edge_feat, params["edge_enc"], 2000)

    d = node.shape[1]
    half = n_nodes // 2
    pad = ((half + 8 + 15) // 16) * 16
    zeros_pad = jnp.zeros((pad, d), F32)

    gather = _gather_kernel(n_nodes, n_edges, d)
    scatter = _scatter_kernel(n_nodes, n_edges, d)

    for bp in params["blocks"]:
        r_rows, s_rows = gather(node.astype(jnp.bfloat16), ridx, sidx)
        proc, edge = _edge_stage(edge, r_rows, s_rows, bp["edge"], 2000)
        agg = scatter(ridx, proc, zeros_pad, bounds2)
        node = _node_stage(node, agg, global_feat, bp["node"], 2000)

    pred = _decoder(node, params["node_dec"], 2000)
    return pred + jnp.asarray(node_size - n_nodes, pred.dtype)
